# Initial kernel scaffold; baseline (speedup 1.0000x reference)
import jax
import jax.numpy as jnp
from jax.experimental import pallas as pl


def kernel(edge_index, rel_type, nid, static_structural, dynamic_structural, coeff1, bases1, loop1, bias1, coeff2, bases2, loop2, bias2, gru_w_ih, gru_w_hh, gru_b_ih, gru_b_hh):
    raise NotImplementedError("write your pallas kernel here")


# speedup vs baseline: 4.4383x; 4.4383x over previous
import functools

import jax
import jax.numpy as jnp
from jax import lax
from jax.experimental import pallas as pl
from jax.experimental.pallas import tpu as pltpu
from jax.experimental.pallas import tpu_sc as plsc

N_NODES = 10000
NUM_RELS = 16
N_EDGES = 320000
D = 128

NC = 2
NS = 16
NW = NC * NS

NROW = 10240
SLAB = NROW // NS
GPT = NROW // NW
EPT = N_EDGES // NW
K = 80
NCH = EPT // K

_mesh = lambda: plsc.VectorSubcoreMesh(
    core_axis_name="c", subcore_axis_name="s", num_cores=NC, num_subcores=NS)


def _sc_entity_gather(nid2, stat, dyn):
    @functools.partial(
        pl.kernel,
        out_type=[jax.ShapeDtypeStruct((NROW, D), jnp.float32),
                  jax.ShapeDtypeStruct((NROW, D), jnp.float32)],
        mesh=_mesh(),
        scratch_types=[
            pltpu.VMEM((GPT,), jnp.int32),
            pltpu.VMEM((K, D), jnp.float32),
            pltpu.VMEM((K, D), jnp.float32),
            pltpu.SemaphoreType.DMA,
            pltpu.SemaphoreType.DMA,
        ],
    )
    def k(nid_h, stat_h, dyn_h, h_o, h0_o, idx_v, rows_v, rows2_v, sem, sem2):
        wid = lax.axis_index("c") * NS + lax.axis_index("s")
        base = wid * GPT
        pltpu.sync_copy(nid_h.at[wid], idx_v)

        def body(i, carry):
            off = base + i * K
            cp1 = pltpu.async_copy(stat_h.at[idx_v.at[pl.ds(i * K, K)]], rows_v, sem)
            cp2 = pltpu.async_copy(dyn_h.at[idx_v.at[pl.ds(i * K, K)]], rows2_v, sem2)
            cp1.wait()
            pltpu.sync_copy(rows_v, h_o.at[pl.ds(off, K)])
            cp2.wait()
            pltpu.sync_copy(rows2_v, h0_o.at[pl.ds(off, K)])
            return carry

        lax.fori_loop(0, GPT // K, body, 0)

    return k(nid2, stat, dyn)


def _sc_edge_agg(ht_flat, gidx3, dst3):
    @functools.partial(
        pl.kernel,
        out_type=jax.ShapeDtypeStruct((NC, NROW, D), jnp.float32),
        mesh=_mesh(),
        scratch_types=[
            pltpu.VMEM((NCH, K), jnp.int32),
            pltpu.VMEM((NCH, K), jnp.int32),
            pltpu.VMEM((K, D), jnp.float32),
            pltpu.VMEM_SHARED((NROW, D), jnp.float32),
            pltpu.SemaphoreType.DMA,
        ],
    )
    def k(ht, gidx_h, dst_h, out, idx2, edst2, rows0, acc, sem0):
        c = lax.axis_index("c")
        s = lax.axis_index("s")
        wid = c * NS + s
        zero16 = jnp.zeros((16,), jnp.float32)

        def zrow(i, carry):
            for j in range(D // 16):
                rows0[i, pl.ds(j * 16, 16)] = zero16
            return carry
        lax.fori_loop(0, K, zrow, 0)

        def zcp(i, carry):
            pltpu.sync_copy(rows0, acc.at[pl.ds(s * SLAB + i * K, K)])
            return carry
        lax.fori_loop(0, SLAB // K, zcp, 0)

        pltpu.sync_copy(gidx_h.at[wid], idx2)
        pltpu.sync_copy(dst_h.at[wid], edst2)

        plsc.subcore_barrier()

        def body(i, carry):
            pltpu.async_copy(ht.at[idx2.at[i]], rows0, sem0).wait()
            pltpu.sync_copy(rows0, acc.at[edst2.at[i]], add=True)
            return carry

        lax.fori_loop(0, NCH, body, 0)

        plsc.subcore_barrier()

        pltpu.sync_copy(acc.at[pl.ds(s * SLAB, SLAB)],
                        out.at[c, pl.ds(s * SLAB, SLAB)])

    return k(ht_flat, gidx3, dst3)


def _sc_degree(dst3):
    @functools.partial(
        pl.kernel,
        out_type=jax.ShapeDtypeStruct((NC, NROW, D), jnp.float32),
        mesh=_mesh(),
        scratch_types=[
            pltpu.VMEM((NCH, K), jnp.int32),
            pltpu.VMEM((K, D), jnp.float32),
            pltpu.VMEM_SHARED((NROW, D), jnp.float32),
        ],
    )
    def k(dst_h, degout, edst2, ones_v, deg_sh):
        c = lax.axis_index("c")
        s = lax.axis_index("s")
        wid = c * NS + s
        zero16 = jnp.zeros((16,), jnp.float32)
        one16 = jnp.ones((16,), jnp.float32)

        def zdrow(i, carry):
            for j in range(D // 16):
                ones_v[i, pl.ds(j * 16, 16)] = zero16
            return carry
        lax.fori_loop(0, K, zdrow, 0)

        def zdcp(i, carry):
            pltpu.sync_copy(ones_v, deg_sh.at[pl.ds(s * SLAB + i * K, K)])
            return carry
        lax.fori_loop(0, SLAB // K, zdcp, 0)

        def onesrow(i, carry):
            ones_v[i, pl.ds(0, 16)] = one16
            return carry
        lax.fori_loop(0, K, onesrow, 0)

        pltpu.sync_copy(dst_h.at[wid], edst2)

        plsc.subcore_barrier()

        def body(i, carry):
            pltpu.sync_copy(ones_v, deg_sh.at[edst2.at[i]], add=True)
            return carry

        lax.fori_loop(0, NCH, body, 0)

        plsc.subcore_barrier()

        pltpu.sync_copy(deg_sh.at[pl.ds(s * SLAB, SLAB)],
                        degout.at[c, pl.ds(s * SLAB, SLAB)])

    return k(dst3)


def _tc_gather_idx(src, rel):
    def body(s_ref, r_ref, o_ref):
        o_ref[...] = r_ref[...] * NROW + s_ref[...]

    return pl.pallas_call(
        body,
        out_shape=jax.ShapeDtypeStruct((N_EDGES // 128, 128), jnp.int32),
    )(src.reshape(N_EDGES // 128, 128),
      rel.reshape(N_EDGES // 128, 128)).reshape(N_EDGES)

def _tc_combine(coeff, bases):
    nb = bases.shape[0]
    bases_f = bases.reshape(nb, D * D)

    def body(c_ref, b_ref, o_ref):
        o_ref[...] = jnp.dot(c_ref[...], b_ref[...],
                             preferred_element_type=jnp.float32)

    w = pl.pallas_call(
        body,
        out_shape=jax.ShapeDtypeStruct((NUM_RELS, D * D), jnp.float32),
    )(coeff, bases_f)
    return w.reshape(NUM_RELS, D, D)


_BN = 512


def _tc_relmm(h, w):
    def body(h_ref, w_ref, o_ref):
        o_ref[0] = jnp.dot(h_ref[...], w_ref[0],
                           preferred_element_type=jnp.float32)

    return pl.pallas_call(
        body,
        grid=(NUM_RELS, NROW // _BN),
        in_specs=[
            pl.BlockSpec((_BN, D), lambda r, n: (n, 0)),
            pl.BlockSpec((1, D, D), lambda r, n: (r, 0, 0)),
        ],
        out_specs=pl.BlockSpec((1, _BN, D), lambda r, n: (r, n, 0)),
        out_shape=jax.ShapeDtypeStruct((NUM_RELS, NROW, D), jnp.float32),
    )(h, w)


def _tc_mix(p, deg16, hprev, loopw, bias):
    def body(p_ref, d_ref, h_ref, w_ref, b_ref, o_ref):
        deg = d_ref[0, :, 0] + d_ref[1, :, 0]
        inv = 1.0 / jnp.maximum(deg, 1.0)
        agg = (p_ref[0] + p_ref[1]) * inv[:, None]
        o_ref[...] = agg + jnp.dot(h_ref[...], w_ref[...],
                                   preferred_element_type=jnp.float32) + b_ref[...]

    return pl.pallas_call(
        body,
        grid=(NROW // _BN,),
        in_specs=[
            pl.BlockSpec((NC, _BN, D), lambda n: (0, n, 0)),
            pl.BlockSpec((NC, _BN, D), lambda n: (0, n, 0)),
            pl.BlockSpec((_BN, D), lambda n: (n, 0)),
            pl.BlockSpec((D, D), lambda n: (0, 0)),
            pl.BlockSpec((1, D), lambda n: (0, 0)),
        ],
        out_specs=pl.BlockSpec((_BN, D), lambda n: (n, 0)),
        out_shape=jax.ShapeDtypeStruct((NROW, D), jnp.float32),
    )(p, deg16, hprev, loopw, bias.reshape(1, D))


def _tc_gru(q, deg16, h1, loopw, bias, h0, w_iht, w_hht, b_ih, b_hh):
    def body(q_ref, d_ref, h1_ref, w_ref, b_ref, h0_ref, wi_ref, wh_ref,
             bi_ref, bh_ref, o_ref):
        deg = d_ref[0, :, 0] + d_ref[1, :, 0]
        inv = 1.0 / jnp.maximum(deg, 1.0)
        x = ((q_ref[0] + q_ref[1]) * inv[:, None]
             + jnp.dot(h1_ref[...], w_ref[...],
                       preferred_element_type=jnp.float32) + b_ref[...])
        h0 = h0_ref[...]
        gi = jnp.dot(x, wi_ref[...], preferred_element_type=jnp.float32) + bi_ref[...]
        gh = jnp.dot(h0, wh_ref[...], preferred_element_type=jnp.float32) + bh_ref[...]
        r = jax.nn.sigmoid(gi[:, :D] + gh[:, :D])
        z = jax.nn.sigmoid(gi[:, D:2 * D] + gh[:, D:2 * D])
        n = jnp.tanh(gi[:, 2 * D:] + r * gh[:, 2 * D:])
        o_ref[...] = (1.0 - z) * n + z * h0

    return pl.pallas_call(
        body,
        grid=(NROW // _BN,),
        in_specs=[
            pl.BlockSpec((NC, _BN, D), lambda n: (0, n, 0)),
            pl.BlockSpec((NC, _BN, D), lambda n: (0, n, 0)),
            pl.BlockSpec((_BN, D), lambda n: (n, 0)),
            pl.BlockSpec((D, D), lambda n: (0, 0)),
            pl.BlockSpec((1, D), lambda n: (0, 0)),
            pl.BlockSpec((_BN, D), lambda n: (n, 0)),
            pl.BlockSpec((D, 3 * D), lambda n: (0, 0)),
            pl.BlockSpec((D, 3 * D), lambda n: (0, 0)),
            pl.BlockSpec((1, 3 * D), lambda n: (0, 0)),
            pl.BlockSpec((1, 3 * D), lambda n: (0, 0)),
        ],
        out_specs=pl.BlockSpec((_BN, D), lambda n: (n, 0)),
        out_shape=jax.ShapeDtypeStruct((NROW, D), jnp.float32),
    )(q, deg16, h1, loopw, bias.reshape(1, D), h0, w_iht, w_hht,
      b_ih.reshape(1, 3 * D), b_hh.reshape(1, 3 * D))


@jax.jit
def kernel(edge_index, rel_type, nid, static_structural, dynamic_structural,
           coeff1, bases1, loop1, bias1, coeff2, bases2, loop2, bias2,
           gru_w_ih, gru_w_hh, gru_b_ih, gru_b_hh):
    gidx3 = _tc_gather_idx(edge_index[0], rel_type).reshape(NW, NCH, K)
    dst3 = edge_index[1].reshape(NW, NCH, K)
    nid2 = jnp.concatenate(
        [nid, jnp.zeros((NROW - N_NODES,), jnp.int32)]).reshape(NW, GPT)
    dyn_flat = dynamic_structural.reshape(-1, D)

    h, h0 = _sc_entity_gather(nid2, static_structural, dyn_flat)

    w1 = _tc_combine(coeff1, bases1)
    ht1 = _tc_relmm(h, w1).reshape(NUM_RELS * NROW, D)
    deg16 = _sc_degree(dst3)
    p1 = _sc_edge_agg(ht1, gidx3, dst3)
    h1 = _tc_mix(p1, deg16, h, loop1, bias1)

    w2 = _tc_combine(coeff2, bases2)
    ht2 = _tc_relmm(h1, w2).reshape(NUM_RELS * NROW, D)
    p2 = _sc_edge_agg(ht2, gidx3, dst3)

    hn = _tc_gru(p2, deg16, h1, loop2, bias2, h0,
                 gru_w_ih.T, gru_w_hh.T, gru_b_ih, gru_b_hh)
    return hn[:N_NODES, None, :]


# speedup vs baseline: 5.4550x; 1.2291x over previous
import functools

import jax
import jax.numpy as jnp
from jax import lax
from jax.experimental import pallas as pl
from jax.experimental.pallas import tpu as pltpu
from jax.experimental.pallas import tpu_sc as plsc

N_NODES = 10000
NUM_RELS = 16
N_EDGES = 320000
D = 128

NC = 2
NS = 16
NW = NC * NS

NROW = 10240
SLAB = NROW // NS
GPT = NROW // NW
EPT = N_EDGES // NW
K = 80
NCH = EPT // K

_mesh = lambda: plsc.VectorSubcoreMesh(
    core_axis_name="c", subcore_axis_name="s", num_cores=NC, num_subcores=NS)


def _sc_entity_gather(nid2, stat, dyn):
    @functools.partial(
        pl.kernel,
        out_type=[jax.ShapeDtypeStruct((NROW, D), jnp.float32),
                  jax.ShapeDtypeStruct((NROW, D), jnp.float32)],
        mesh=_mesh(),
        scratch_types=[
            pltpu.VMEM((GPT,), jnp.int32),
            pltpu.VMEM((K, D), jnp.float32),
            pltpu.VMEM((K, D), jnp.float32),
            pltpu.SemaphoreType.DMA,
            pltpu.SemaphoreType.DMA,
        ],
    )
    def k(nid_h, stat_h, dyn_h, h_o, h0_o, idx_v, rows_v, rows2_v, sem, sem2):
        wid = lax.axis_index("c") * NS + lax.axis_index("s")
        base = wid * GPT
        pltpu.sync_copy(nid_h.at[wid], idx_v)

        def body(i, carry):
            off = base + i * K
            cp1 = pltpu.async_copy(stat_h.at[idx_v.at[pl.ds(i * K, K)]], rows_v, sem)
            cp2 = pltpu.async_copy(dyn_h.at[idx_v.at[pl.ds(i * K, K)]], rows2_v, sem2)
            cp1.wait()
            pltpu.sync_copy(rows_v, h_o.at[pl.ds(off, K)])
            cp2.wait()
            pltpu.sync_copy(rows2_v, h0_o.at[pl.ds(off, K)])
            return carry

        lax.fori_loop(0, GPT // K, body, 0)

    return k(nid2, stat, dyn)


def _sc_edge_agg(ht_flat, gidx2, dst3):
    @functools.partial(
        pl.kernel,
        out_type=jax.ShapeDtypeStruct((NC, NROW, D), jnp.float32),
        mesh=_mesh(),
        scratch_types=[
            pltpu.VMEM((EPT,), jnp.int32),
            pltpu.VMEM((NCH, K), jnp.int32),
            pltpu.VMEM((K, D), jnp.float32),
            pltpu.VMEM((K, D), jnp.float32),
            pltpu.VMEM_SHARED((NROW, D), jnp.float32),
            pltpu.SemaphoreType.DMA,
            pltpu.SemaphoreType.DMA,
        ],
    )
    def k(ht, gidx_h, dst_h, out, idx2, edst2, rows0, rows1, acc, sem0, sem1):
        c = lax.axis_index("c")
        s = lax.axis_index("s")
        wid = c * NS + s
        zero16 = jnp.zeros((16,), jnp.float32)

        def zrow(i, carry):
            for j in range(D // 16):
                rows0[i, pl.ds(j * 16, 16)] = zero16
            return carry
        lax.fori_loop(0, K, zrow, 0)

        def zcp(i, carry):
            pltpu.sync_copy(rows0, acc.at[pl.ds(s * SLAB + i * K, K)])
            return carry
        lax.fori_loop(0, SLAB // K, zcp, 0)

        pltpu.sync_copy(gidx_h.at[wid], idx2)
        pltpu.sync_copy(dst_h.at[wid], edst2)

        plsc.subcore_barrier()

        pltpu.async_copy(ht.at[idx2.at[pl.ds(0, K)]], rows0, sem0)

        def body(t, carry):
            a = 2 * t
            pltpu.async_copy(ht.at[idx2.at[pl.ds((a + 1) * K, K)]], rows1, sem1)
            pltpu.make_async_copy(ht.at[idx2.at[pl.ds(a * K, K)]], rows0, sem0).wait()
            pltpu.sync_copy(rows0, acc.at[edst2.at[a]], add=True)
            pltpu.async_copy(ht.at[idx2.at[pl.ds((a + 2) * K, K)]], rows0, sem0)
            pltpu.make_async_copy(ht.at[idx2.at[pl.ds((a + 1) * K, K)]], rows1, sem1).wait()
            pltpu.sync_copy(rows1, acc.at[edst2.at[a + 1]], add=True)
            return carry

        lax.fori_loop(0, (NCH - 1) // 2, body, 0)
        pltpu.make_async_copy(ht.at[idx2.at[pl.ds((NCH - 1) * K, K)]], rows0, sem0).wait()
        pltpu.sync_copy(rows0, acc.at[edst2.at[NCH - 1]], add=True)

        plsc.subcore_barrier()

        pltpu.sync_copy(acc.at[pl.ds(s * SLAB, SLAB)],
                        out.at[c, pl.ds(s * SLAB, SLAB)])

    return k(ht_flat, gidx2, dst3)


def _sc_degree(dst3):
    @functools.partial(
        pl.kernel,
        out_type=jax.ShapeDtypeStruct((NC, NROW, D), jnp.float32),
        mesh=_mesh(),
        scratch_types=[
            pltpu.VMEM((NCH, K), jnp.int32),
            pltpu.VMEM((K, D), jnp.float32),
            pltpu.VMEM_SHARED((NROW, D), jnp.float32),
        ],
    )
    def k(dst_h, degout, edst2, ones_v, deg_sh):
        c = lax.axis_index("c")
        s = lax.axis_index("s")
        wid = c * NS + s
        zero16 = jnp.zeros((16,), jnp.float32)
        one16 = jnp.ones((16,), jnp.float32)

        def zdrow(i, carry):
            for j in range(D // 16):
                ones_v[i, pl.ds(j * 16, 16)] = zero16
            return carry
        lax.fori_loop(0, K, zdrow, 0)

        def zdcp(i, carry):
            pltpu.sync_copy(ones_v, deg_sh.at[pl.ds(s * SLAB + i * K, K)])
            return carry
        lax.fori_loop(0, SLAB // K, zdcp, 0)

        def onesrow(i, carry):
            ones_v[i, pl.ds(0, 16)] = one16
            return carry
        lax.fori_loop(0, K, onesrow, 0)

        pltpu.sync_copy(dst_h.at[wid], edst2)

        plsc.subcore_barrier()

        def body(i, carry):
            pltpu.sync_copy(ones_v, deg_sh.at[edst2.at[i]], add=True)
            return carry

        lax.fori_loop(0, NCH, body, 0)

        plsc.subcore_barrier()

        pltpu.sync_copy(deg_sh.at[pl.ds(s * SLAB, SLAB)],
                        degout.at[c, pl.ds(s * SLAB, SLAB)])

    return k(dst3)


def _tc_gather_idx(src, rel):
    def body(s_ref, r_ref, o_ref):
        o_ref[...] = r_ref[...] * NROW + s_ref[...]

    return pl.pallas_call(
        body,
        out_shape=jax.ShapeDtypeStruct((N_EDGES // 128, 128), jnp.int32),
    )(src.reshape(N_EDGES // 128, 128),
      rel.reshape(N_EDGES // 128, 128)).reshape(N_EDGES)

def _tc_combine(coeff, bases):
    nb = bases.shape[0]
    bases_f = bases.reshape(nb, D * D)

    def body(c_ref, b_ref, o_ref):
        o_ref[...] = jnp.dot(c_ref[...], b_ref[...],
                             preferred_element_type=jnp.float32)

    w = pl.pallas_call(
        body,
        out_shape=jax.ShapeDtypeStruct((NUM_RELS, D * D), jnp.float32),
    )(coeff, bases_f)
    return w.reshape(NUM_RELS, D, D)


_BN = 512


def _tc_relmm(h, w):
    def body(h_ref, w_ref, o_ref):
        o_ref[0] = jnp.dot(h_ref[...], w_ref[0],
                           preferred_element_type=jnp.float32)

    return pl.pallas_call(
        body,
        grid=(NUM_RELS, NROW // _BN),
        in_specs=[
            pl.BlockSpec((_BN, D), lambda r, n: (n, 0)),
            pl.BlockSpec((1, D, D), lambda r, n: (r, 0, 0)),
        ],
        out_specs=pl.BlockSpec((1, _BN, D), lambda r, n: (r, n, 0)),
        out_shape=jax.ShapeDtypeStruct((NUM_RELS, NROW, D), jnp.float32),
    )(h, w)


def _tc_mix(p, deg16, hprev, loopw, bias):
    def body(p_ref, d_ref, h_ref, w_ref, b_ref, o_ref):
        deg = d_ref[0, :, 0] + d_ref[1, :, 0]
        inv = 1.0 / jnp.maximum(deg, 1.0)
        agg = (p_ref[0] + p_ref[1]) * inv[:, None]
        o_ref[...] = agg + jnp.dot(h_ref[...], w_ref[...],
                                   preferred_element_type=jnp.float32) + b_ref[...]

    return pl.pallas_call(
        body,
        grid=(NROW // _BN,),
        in_specs=[
            pl.BlockSpec((NC, _BN, D), lambda n: (0, n, 0)),
            pl.BlockSpec((NC, _BN, D), lambda n: (0, n, 0)),
            pl.BlockSpec((_BN, D), lambda n: (n, 0)),
            pl.BlockSpec((D, D), lambda n: (0, 0)),
            pl.BlockSpec((1, D), lambda n: (0, 0)),
        ],
        out_specs=pl.BlockSpec((_BN, D), lambda n: (n, 0)),
        out_shape=jax.ShapeDtypeStruct((NROW, D), jnp.float32),
    )(p, deg16, hprev, loopw, bias.reshape(1, D))


def _tc_gru(q, deg16, h1, loopw, bias, h0, w_iht, w_hht, b_ih, b_hh):
    def body(q_ref, d_ref, h1_ref, w_ref, b_ref, h0_ref, wi_ref, wh_ref,
             bi_ref, bh_ref, o_ref):
        deg = d_ref[0, :, 0] + d_ref[1, :, 0]
        inv = 1.0 / jnp.maximum(deg, 1.0)
        x = ((q_ref[0] + q_ref[1]) * inv[:, None]
             + jnp.dot(h1_ref[...], w_ref[...],
                       preferred_element_type=jnp.float32) + b_ref[...])
        h0 = h0_ref[...]
        gi = jnp.dot(x, wi_ref[...], preferred_element_type=jnp.float32) + bi_ref[...]
        gh = jnp.dot(h0, wh_ref[...], preferred_element_type=jnp.float32) + bh_ref[...]
        r = jax.nn.sigmoid(gi[:, :D] + gh[:, :D])
        z = jax.nn.sigmoid(gi[:, D:2 * D] + gh[:, D:2 * D])
        n = jnp.tanh(gi[:, 2 * D:] + r * gh[:, 2 * D:])
        o_ref[...] = (1.0 - z) * n + z * h0

    return pl.pallas_call(
        body,
        grid=(NROW // _BN,),
        in_specs=[
            pl.BlockSpec((NC, _BN, D), lambda n: (0, n, 0)),
            pl.BlockSpec((NC, _BN, D), lambda n: (0, n, 0)),
            pl.BlockSpec((_BN, D), lambda n: (n, 0)),
            pl.BlockSpec((D, D), lambda n: (0, 0)),
            pl.BlockSpec((1, D), lambda n: (0, 0)),
            pl.BlockSpec((_BN, D), lambda n: (n, 0)),
            pl.BlockSpec((D, 3 * D), lambda n: (0, 0)),
            pl.BlockSpec((D, 3 * D), lambda n: (0, 0)),
            pl.BlockSpec((1, 3 * D), lambda n: (0, 0)),
            pl.BlockSpec((1, 3 * D), lambda n: (0, 0)),
        ],
        out_specs=pl.BlockSpec((_BN, D), lambda n: (n, 0)),
        out_shape=jax.ShapeDtypeStruct((NROW, D), jnp.float32),
    )(q, deg16, h1, loopw, bias.reshape(1, D), h0, w_iht, w_hht,
      b_ih.reshape(1, 3 * D), b_hh.reshape(1, 3 * D))


@jax.jit
def kernel(edge_index, rel_type, nid, static_structural, dynamic_structural,
           coeff1, bases1, loop1, bias1, coeff2, bases2, loop2, bias2,
           gru_w_ih, gru_w_hh, gru_b_ih, gru_b_hh):
    gidx2 = _tc_gather_idx(edge_index[0], rel_type).reshape(NW, EPT)
    dst3 = edge_index[1].reshape(NW, NCH, K)
    nid2 = jnp.concatenate(
        [nid, jnp.zeros((NROW - N_NODES,), jnp.int32)]).reshape(NW, GPT)
    dyn_flat = dynamic_structural.reshape(-1, D)

    h, h0 = _sc_entity_gather(nid2, static_structural, dyn_flat)

    w1 = _tc_combine(coeff1, bases1)
    ht1 = _tc_relmm(h, w1).reshape(NUM_RELS * NROW, D)
    deg16 = _sc_degree(dst3)
    p1 = _sc_edge_agg(ht1, gidx2, dst3)
    h1 = _tc_mix(p1, deg16, h, loop1, bias1)

    w2 = _tc_combine(coeff2, bases2)
    ht2 = _tc_relmm(h1, w2).reshape(NUM_RELS * NROW, D)
    p2 = _sc_edge_agg(ht2, gidx2, dst3)

    hn = _tc_gru(p2, deg16, h1, loop2, bias2, h0,
                 gru_w_ih.T, gru_w_hh.T, gru_b_ih, gru_b_hh)
    return hn[:N_NODES, None, :]


# speedup vs baseline: 9.4113x; 1.7252x over previous
import functools

import jax
import jax.numpy as jnp
from jax import lax
from jax.experimental import pallas as pl
from jax.experimental.pallas import tpu as pltpu
from jax.experimental.pallas import tpu_sc as plsc

N_NODES = 10000
NUM_RELS = 16
N_EDGES = 320000
D = 128

NC = 2
NS = 16
NW = NC * NS

NROW = 10240
SLAB = NROW // NS
GPT = NROW // NW
EPT = N_EDGES // NW
K = 80
NCH = EPT // K

_mesh = lambda: plsc.VectorSubcoreMesh(
    core_axis_name="c", subcore_axis_name="s", num_cores=NC, num_subcores=NS)


def _sc_entity_gather(nid2, stat, dyn):
    @functools.partial(
        pl.kernel,
        out_type=[jax.ShapeDtypeStruct((NROW, D), jnp.float32),
                  jax.ShapeDtypeStruct((NROW, D), jnp.float32)],
        mesh=_mesh(),
        scratch_types=[
            pltpu.VMEM((GPT,), jnp.int32),
            pltpu.VMEM((K, D), jnp.float32),
            pltpu.VMEM((K, D), jnp.float32),
            pltpu.SemaphoreType.DMA,
            pltpu.SemaphoreType.DMA,
        ],
    )
    def k(nid_h, stat_h, dyn_h, h_o, h0_o, idx_v, rows_v, rows2_v, sem, sem2):
        wid = lax.axis_index("c") * NS + lax.axis_index("s")
        base = wid * GPT
        pltpu.sync_copy(nid_h.at[wid], idx_v)

        def body(i, carry):
            off = base + i * K
            cp1 = pltpu.async_copy(stat_h.at[idx_v.at[pl.ds(i * K, K)]], rows_v, sem)
            cp2 = pltpu.async_copy(dyn_h.at[idx_v.at[pl.ds(i * K, K)]], rows2_v, sem2)
            cp1.wait()
            pltpu.sync_copy(rows_v, h_o.at[pl.ds(off, K)])
            cp2.wait()
            pltpu.sync_copy(rows2_v, h0_o.at[pl.ds(off, K)])
            return carry

        lax.fori_loop(0, GPT // K, body, 0)

    return k(nid2, stat, dyn)


def _sc_edge_agg(ht_flat, gidx2, dst3):
    @functools.partial(
        pl.kernel,
        out_type=jax.ShapeDtypeStruct((NC, NROW, D), jnp.float32),
        mesh=_mesh(),
        scratch_types=[
            pltpu.VMEM((EPT,), jnp.int32),
            pltpu.VMEM((NCH, K), jnp.int32),
            pltpu.VMEM((K, D), jnp.float32),
            pltpu.VMEM((K, D), jnp.float32),
            pltpu.VMEM_SHARED((NROW, D), jnp.float32),
            pltpu.SemaphoreType.DMA,
            pltpu.SemaphoreType.DMA,
        ],
    )
    def k(ht, gidx_h, dst_h, out, idx2, edst2, rows0, rows1, acc, sem0, sem1):
        c = lax.axis_index("c")
        s = lax.axis_index("s")
        wid = c * NS + s
        zero16 = jnp.zeros((16,), jnp.float32)

        def zrow(i, carry):
            for j in range(D // 16):
                rows0[i, pl.ds(j * 16, 16)] = zero16
            return carry
        lax.fori_loop(0, K, zrow, 0)

        def zcp(i, carry):
            pltpu.sync_copy(rows0, acc.at[pl.ds(s * SLAB + i * K, K)])
            return carry
        lax.fori_loop(0, SLAB // K, zcp, 0)

        pltpu.sync_copy(gidx_h.at[wid], idx2)
        pltpu.sync_copy(dst_h.at[wid], edst2)

        plsc.subcore_barrier()

        pltpu.async_copy(ht.at[idx2.at[pl.ds(0, K)]], rows0, sem0)

        def body(t, carry):
            a = 2 * t
            pltpu.async_copy(ht.at[idx2.at[pl.ds((a + 1) * K, K)]], rows1, sem1)
            pltpu.make_async_copy(ht.at[idx2.at[pl.ds(a * K, K)]], rows0, sem0).wait()
            pltpu.sync_copy(rows0, acc.at[edst2.at[a]], add=True)
            pltpu.async_copy(ht.at[idx2.at[pl.ds((a + 2) * K, K)]], rows0, sem0)
            pltpu.make_async_copy(ht.at[idx2.at[pl.ds((a + 1) * K, K)]], rows1, sem1).wait()
            pltpu.sync_copy(rows1, acc.at[edst2.at[a + 1]], add=True)
            return carry

        lax.fori_loop(0, (NCH - 1) // 2, body, 0)
        pltpu.make_async_copy(ht.at[idx2.at[pl.ds((NCH - 1) * K, K)]], rows0, sem0).wait()
        pltpu.sync_copy(rows0, acc.at[edst2.at[NCH - 1]], add=True)

        plsc.subcore_barrier()

        pltpu.sync_copy(acc.at[pl.ds(s * SLAB, SLAB)],
                        out.at[c, pl.ds(s * SLAB, SLAB)])

    return k(ht_flat, gidx2, dst3)


def _sc_degree(dst3):
    @functools.partial(
        pl.kernel,
        out_type=jax.ShapeDtypeStruct((NC, NROW, D), jnp.float32),
        mesh=_mesh(),
        scratch_types=[
            pltpu.VMEM((NCH, K), jnp.int32),
            pltpu.VMEM((K, D), jnp.float32),
            pltpu.VMEM_SHARED((NROW, D), jnp.float32),
        ],
    )
    def k(dst_h, degout, edst2, ones_v, deg_sh):
        c = lax.axis_index("c")
        s = lax.axis_index("s")
        wid = c * NS + s
        zero16 = jnp.zeros((16,), jnp.float32)
        one16 = jnp.ones((16,), jnp.float32)

        def zdrow(i, carry):
            for j in range(D // 16):
                ones_v[i, pl.ds(j * 16, 16)] = zero16
            return carry
        lax.fori_loop(0, K, zdrow, 0)

        def zdcp(i, carry):
            pltpu.sync_copy(ones_v, deg_sh.at[pl.ds(s * SLAB + i * K, K)])
            return carry
        lax.fori_loop(0, SLAB // K, zdcp, 0)

        def onesrow(i, carry):
            ones_v[i, pl.ds(0, 16)] = one16
            return carry
        lax.fori_loop(0, K, onesrow, 0)

        pltpu.sync_copy(dst_h.at[wid], edst2)

        plsc.subcore_barrier()

        def body(i, carry):
            pltpu.sync_copy(ones_v, deg_sh.at[edst2.at[i]], add=True)
            return carry

        lax.fori_loop(0, NCH, body, 0)

        plsc.subcore_barrier()

        pltpu.sync_copy(deg_sh.at[pl.ds(s * SLAB, SLAB)],
                        degout.at[c, pl.ds(s * SLAB, SLAB)])

    return k(dst3)


def _tc_gather_idx(src, rel):
    def body(s_ref, r_ref, o_ref):
        o_ref[...] = r_ref[...] * NROW + s_ref[...]

    return pl.pallas_call(
        body,
        out_shape=jax.ShapeDtypeStruct((N_EDGES // 128, 128), jnp.int32),
    )(src.reshape(N_EDGES // 128, 128),
      rel.reshape(N_EDGES // 128, 128)).reshape(N_EDGES)

def _tc_combine(coeff, bases):
    nb = bases.shape[0]
    bases_f = bases.reshape(nb, D * D)

    def body(c_ref, b_ref, o_ref):
        o_ref[...] = jnp.dot(c_ref[...], b_ref[...],
                             preferred_element_type=jnp.float32)

    w = pl.pallas_call(
        body,
        out_shape=jax.ShapeDtypeStruct((NUM_RELS, D * D), jnp.float32),
    )(coeff, bases_f)
    return w.reshape(NUM_RELS, D, D)


_BN = 512


def _tc_relmm(h, w):
    def body(h_ref, w_ref, o_ref):
        hb = h_ref[...]
        for r in range(NUM_RELS):
            o_ref[r] = jnp.dot(hb, w_ref[r], preferred_element_type=jnp.float32)

    return pl.pallas_call(
        body,
        grid=(NROW // _BN,),
        in_specs=[
            pl.BlockSpec((_BN, D), lambda n: (n, 0)),
            pl.BlockSpec((NUM_RELS, D, D), lambda n: (0, 0, 0)),
        ],
        out_specs=pl.BlockSpec((NUM_RELS, _BN, D), lambda n: (0, n, 0)),
        out_shape=jax.ShapeDtypeStruct((NUM_RELS, NROW, D), jnp.float32),
    )(h, w)


def _tc_mix_relmm(p, deg, hprev, loopw, bias, w2):
    def body(p_ref, d_ref, h_ref, w_ref, b_ref, w2_ref, h1_ref, o_ref):
        dg = d_ref[0, :, 0] + d_ref[1, :, 0]
        inv = 1.0 / jnp.maximum(dg, 1.0)
        h1 = ((p_ref[0] + p_ref[1]) * inv[:, None]
              + jnp.dot(h_ref[...], w_ref[...],
                        preferred_element_type=jnp.float32) + b_ref[...])
        h1_ref[...] = h1
        for r in range(NUM_RELS):
            o_ref[r] = jnp.dot(h1, w2_ref[r], preferred_element_type=jnp.float32)

    return pl.pallas_call(
        body,
        grid=(NROW // _BN,),
        in_specs=[
            pl.BlockSpec((NC, _BN, D), lambda n: (0, n, 0)),
            pl.BlockSpec((NC, _BN, D), lambda n: (0, n, 0)),
            pl.BlockSpec((_BN, D), lambda n: (n, 0)),
            pl.BlockSpec((D, D), lambda n: (0, 0)),
            pl.BlockSpec((1, D), lambda n: (0, 0)),
            pl.BlockSpec((NUM_RELS, D, D), lambda n: (0, 0, 0)),
        ],
        out_specs=[
            pl.BlockSpec((_BN, D), lambda n: (n, 0)),
            pl.BlockSpec((NUM_RELS, _BN, D), lambda n: (0, n, 0)),
        ],
        out_shape=[
            jax.ShapeDtypeStruct((NROW, D), jnp.float32),
            jax.ShapeDtypeStruct((NUM_RELS, NROW, D), jnp.float32),
        ],
    )(p, deg, hprev, loopw, bias.reshape(1, D), w2)


def _tc_mix(p, deg16, hprev, loopw, bias):
    def body(p_ref, d_ref, h_ref, w_ref, b_ref, o_ref):
        deg = d_ref[0, :, 0] + d_ref[1, :, 0]
        inv = 1.0 / jnp.maximum(deg, 1.0)
        agg = (p_ref[0] + p_ref[1]) * inv[:, None]
        o_ref[...] = agg + jnp.dot(h_ref[...], w_ref[...],
                                   preferred_element_type=jnp.float32) + b_ref[...]

    return pl.pallas_call(
        body,
        grid=(NROW // _BN,),
        in_specs=[
            pl.BlockSpec((NC, _BN, D), lambda n: (0, n, 0)),
            pl.BlockSpec((NC, _BN, D), lambda n: (0, n, 0)),
            pl.BlockSpec((_BN, D), lambda n: (n, 0)),
            pl.BlockSpec((D, D), lambda n: (0, 0)),
            pl.BlockSpec((1, D), lambda n: (0, 0)),
        ],
        out_specs=pl.BlockSpec((_BN, D), lambda n: (n, 0)),
        out_shape=jax.ShapeDtypeStruct((NROW, D), jnp.float32),
    )(p, deg16, hprev, loopw, bias.reshape(1, D))


def _tc_gru(q, deg16, h1, loopw, bias, h0, w_iht, w_hht, b_ih, b_hh):
    def body(q_ref, d_ref, h1_ref, w_ref, b_ref, h0_ref, wi_ref, wh_ref,
             bi_ref, bh_ref, o_ref):
        deg = d_ref[0, :, 0] + d_ref[1, :, 0]
        inv = 1.0 / jnp.maximum(deg, 1.0)
        x = ((q_ref[0] + q_ref[1]) * inv[:, None]
             + jnp.dot(h1_ref[...], w_ref[...],
                       preferred_element_type=jnp.float32) + b_ref[...])
        h0 = h0_ref[...]
        gi = jnp.dot(x, wi_ref[...], preferred_element_type=jnp.float32) + bi_ref[...]
        gh = jnp.dot(h0, wh_ref[...], preferred_element_type=jnp.float32) + bh_ref[...]
        r = jax.nn.sigmoid(gi[:, :D] + gh[:, :D])
        z = jax.nn.sigmoid(gi[:, D:2 * D] + gh[:, D:2 * D])
        n = jnp.tanh(gi[:, 2 * D:] + r * gh[:, 2 * D:])
        o_ref[...] = (1.0 - z) * n + z * h0

    return pl.pallas_call(
        body,
        grid=(NROW // _BN,),
        in_specs=[
            pl.BlockSpec((NC, _BN, D), lambda n: (0, n, 0)),
            pl.BlockSpec((NC, _BN, D), lambda n: (0, n, 0)),
            pl.BlockSpec((_BN, D), lambda n: (n, 0)),
            pl.BlockSpec((D, D), lambda n: (0, 0)),
            pl.BlockSpec((1, D), lambda n: (0, 0)),
            pl.BlockSpec((_BN, D), lambda n: (n, 0)),
            pl.BlockSpec((D, 3 * D), lambda n: (0, 0)),
            pl.BlockSpec((D, 3 * D), lambda n: (0, 0)),
            pl.BlockSpec((1, 3 * D), lambda n: (0, 0)),
            pl.BlockSpec((1, 3 * D), lambda n: (0, 0)),
        ],
        out_specs=pl.BlockSpec((_BN, D), lambda n: (n, 0)),
        out_shape=jax.ShapeDtypeStruct((NROW, D), jnp.float32),
    )(q, deg16, h1, loopw, bias.reshape(1, D), h0, w_iht, w_hht,
      b_ih.reshape(1, 3 * D), b_hh.reshape(1, 3 * D))


@jax.jit
def kernel(edge_index, rel_type, nid, static_structural, dynamic_structural,
           coeff1, bases1, loop1, bias1, coeff2, bases2, loop2, bias2,
           gru_w_ih, gru_w_hh, gru_b_ih, gru_b_hh):
    gidx2 = _tc_gather_idx(edge_index[0], rel_type).reshape(NW, EPT)
    dst3 = edge_index[1].reshape(NW, NCH, K)
    nid2 = jnp.concatenate(
        [nid, jnp.zeros((NROW - N_NODES,), jnp.int32)]).reshape(NW, GPT)
    dyn_flat = dynamic_structural.reshape(-1, D)

    h, h0 = _sc_entity_gather(nid2, static_structural, dyn_flat)

    w1 = _tc_combine(coeff1, bases1)
    ht1 = _tc_relmm(h, w1).reshape(NUM_RELS * NROW, D)
    deg16 = _sc_degree(dst3)
    p1 = _sc_edge_agg(ht1, gidx2, dst3)

    w2 = _tc_combine(coeff2, bases2)
    h1, ht2 = _tc_mix_relmm(p1, deg16, h, loop1, bias1, w2)
    ht2 = ht2.reshape(NUM_RELS * NROW, D)
    p2 = _sc_edge_agg(ht2, gidx2, dst3)

    hn = _tc_gru(p2, deg16, h1, loop2, bias2, h0,
                 gru_w_ih.T, gru_w_hh.T, gru_b_ih, gru_b_hh)
    return hn[:N_NODES, None, :]


# speedup vs baseline: 9.5432x; 1.0140x over previous
import functools

import jax
import jax.numpy as jnp
from jax import lax
from jax.experimental import pallas as pl
from jax.experimental.pallas import tpu as pltpu
from jax.experimental.pallas import tpu_sc as plsc

N_NODES = 10000
NUM_RELS = 16
N_EDGES = 320000
D = 128

NC = 2
NS = 16
NW = NC * NS

NROW = 10240
SLAB = NROW // NS
GPT = NROW // NW
EPT = N_EDGES // NW
K = 80
NCH = EPT // K

_mesh = lambda: plsc.VectorSubcoreMesh(
    core_axis_name="c", subcore_axis_name="s", num_cores=NC, num_subcores=NS)


def _sc_entity_gather(nid2, stat, dyn):
    @functools.partial(
        pl.kernel,
        out_type=[jax.ShapeDtypeStruct((NROW, D), jnp.float32),
                  jax.ShapeDtypeStruct((NROW, D), jnp.float32)],
        mesh=_mesh(),
        scratch_types=[
            pltpu.VMEM((GPT,), jnp.int32),
            pltpu.VMEM((K, D), jnp.float32),
            pltpu.VMEM((K, D), jnp.float32),
            pltpu.SemaphoreType.DMA,
            pltpu.SemaphoreType.DMA,
        ],
    )
    def k(nid_h, stat_h, dyn_h, h_o, h0_o, idx_v, rows_v, rows2_v, sem, sem2):
        wid = lax.axis_index("c") * NS + lax.axis_index("s")
        base = wid * GPT
        pltpu.sync_copy(nid_h.at[wid], idx_v)

        def body(i, carry):
            off = base + i * K
            cp1 = pltpu.async_copy(stat_h.at[idx_v.at[pl.ds(i * K, K)]], rows_v, sem)
            cp2 = pltpu.async_copy(dyn_h.at[idx_v.at[pl.ds(i * K, K)]], rows2_v, sem2)
            cp1.wait()
            pltpu.sync_copy(rows_v, h_o.at[pl.ds(off, K)])
            cp2.wait()
            pltpu.sync_copy(rows2_v, h0_o.at[pl.ds(off, K)])
            return carry

        lax.fori_loop(0, GPT // K, body, 0)

    return k(nid2, stat, dyn)


def _sc_edge_agg(ht_flat, gidx2, dst3):
    @functools.partial(
        pl.kernel,
        out_type=jax.ShapeDtypeStruct((NC, NROW, D), jnp.float32),
        mesh=_mesh(),
        scratch_types=[
            pltpu.VMEM((EPT,), jnp.int32),
            pltpu.VMEM((NCH, K), jnp.int32),
            pltpu.VMEM((K, D), jnp.float32),
            pltpu.VMEM((K, D), jnp.float32),
            pltpu.VMEM_SHARED((NROW, D), jnp.float32),
            pltpu.SemaphoreType.DMA,
            pltpu.SemaphoreType.DMA,
        ],
    )
    def k(ht, gidx_h, dst_h, out, idx2, edst2, rows0, rows1, acc, sem0, sem1):
        c = lax.axis_index("c")
        s = lax.axis_index("s")
        wid = c * NS + s
        zero16 = jnp.zeros((16,), jnp.float32)

        def zrow(i, carry):
            for j in range(D // 16):
                rows0[i, pl.ds(j * 16, 16)] = zero16
            return carry
        lax.fori_loop(0, K, zrow, 0)

        def zcp(i, carry):
            pltpu.sync_copy(rows0, acc.at[pl.ds(s * SLAB + i * K, K)])
            return carry
        lax.fori_loop(0, SLAB // K, zcp, 0)

        pltpu.sync_copy(gidx_h.at[wid], idx2)
        pltpu.sync_copy(dst_h.at[wid], edst2)

        plsc.subcore_barrier()

        pltpu.async_copy(ht.at[idx2.at[pl.ds(0, K)]], rows0, sem0)

        def body(t, carry):
            a = 2 * t
            pltpu.async_copy(ht.at[idx2.at[pl.ds((a + 1) * K, K)]], rows1, sem1)
            pltpu.make_async_copy(ht.at[idx2.at[pl.ds(a * K, K)]], rows0, sem0).wait()
            pltpu.sync_copy(rows0, acc.at[edst2.at[a]], add=True)
            pltpu.async_copy(ht.at[idx2.at[pl.ds((a + 2) * K, K)]], rows0, sem0)
            pltpu.make_async_copy(ht.at[idx2.at[pl.ds((a + 1) * K, K)]], rows1, sem1).wait()
            pltpu.sync_copy(rows1, acc.at[edst2.at[a + 1]], add=True)
            return carry

        lax.fori_loop(0, (NCH - 1) // 2, body, 0)
        pltpu.make_async_copy(ht.at[idx2.at[pl.ds((NCH - 1) * K, K)]], rows0, sem0).wait()
        pltpu.sync_copy(rows0, acc.at[edst2.at[NCH - 1]], add=True)

        plsc.subcore_barrier()

        pltpu.sync_copy(acc.at[pl.ds(s * SLAB, SLAB)],
                        out.at[c, pl.ds(s * SLAB, SLAB)])

    return k(ht_flat, gidx2, dst3)


def _sc_degree(dst3):
    @functools.partial(
        pl.kernel,
        out_type=jax.ShapeDtypeStruct((NC, NROW, D), jnp.float32),
        mesh=_mesh(),
        scratch_types=[
            pltpu.VMEM((NCH, K), jnp.int32),
            pltpu.VMEM((K, D), jnp.float32),
            pltpu.VMEM_SHARED((NROW, D), jnp.float32),
        ],
    )
    def k(dst_h, degout, edst2, ones_v, deg_sh):
        c = lax.axis_index("c")
        s = lax.axis_index("s")
        wid = c * NS + s
        zero16 = jnp.zeros((16,), jnp.float32)
        one16 = jnp.ones((16,), jnp.float32)

        def zdrow(i, carry):
            for j in range(D // 16):
                ones_v[i, pl.ds(j * 16, 16)] = zero16
            return carry
        lax.fori_loop(0, K, zdrow, 0)

        def zdcp(i, carry):
            pltpu.sync_copy(ones_v, deg_sh.at[pl.ds(s * SLAB + i * K, K)])
            return carry
        lax.fori_loop(0, SLAB // K, zdcp, 0)

        def onesrow(i, carry):
            ones_v[i, pl.ds(0, 16)] = one16
            return carry
        lax.fori_loop(0, K, onesrow, 0)

        pltpu.sync_copy(dst_h.at[wid], edst2)

        plsc.subcore_barrier()

        def body(i, carry):
            pltpu.sync_copy(ones_v, deg_sh.at[edst2.at[i]], add=True)
            return carry

        lax.fori_loop(0, NCH, body, 0)

        plsc.subcore_barrier()

        pltpu.sync_copy(deg_sh.at[pl.ds(s * SLAB, SLAB)],
                        degout.at[c, pl.ds(s * SLAB, SLAB)])

    return k(dst3)


def _tc_gather_idx(src, rel):
    def body(s_ref, r_ref, o_ref):
        o_ref[...] = r_ref[...] * NROW + s_ref[...]

    return pl.pallas_call(
        body,
        out_shape=jax.ShapeDtypeStruct((N_EDGES // 128, 128), jnp.int32),
    )(src.reshape(N_EDGES // 128, 128),
      rel.reshape(N_EDGES // 128, 128)).reshape(N_EDGES)

def _tc_combine(coeff, bases):
    nb = bases.shape[0]
    bases_f = bases.reshape(nb, D * D)

    def body(c_ref, b_ref, o_ref):
        o_ref[...] = jnp.dot(c_ref[...], b_ref[...],
                             preferred_element_type=jnp.float32)

    w = pl.pallas_call(
        body,
        out_shape=jax.ShapeDtypeStruct((NUM_RELS, D * D), jnp.float32),
    )(coeff, bases_f)
    return w.reshape(NUM_RELS, D, D)


_BN = 512


def _tc_relmm(h, w):
    def body(h_ref, w_ref, o_ref):
        hb = h_ref[...]
        for r in range(NUM_RELS):
            o_ref[r] = jnp.dot(hb, w_ref[r], preferred_element_type=jnp.float32)

    return pl.pallas_call(
        body,
        grid=(NROW // _BN,),
        in_specs=[
            pl.BlockSpec((_BN, D), lambda n: (n, 0)),
            pl.BlockSpec((NUM_RELS, D, D), lambda n: (0, 0, 0)),
        ],
        out_specs=pl.BlockSpec((NUM_RELS, _BN, D), lambda n: (0, n, 0)),
        out_shape=jax.ShapeDtypeStruct((NUM_RELS, NROW, D), jnp.float32),
    )(h, w)


def _tc_mix_relmm(p, deg, hprev, loopw, bias, w2):
    def body(p_ref, d_ref, h_ref, w_ref, b_ref, w2_ref, h1_ref, o_ref,
             inv_ref):
        dg = d_ref[0, :, 0] + d_ref[1, :, 0]
        inv = 1.0 / jnp.maximum(dg, 1.0)
        inv_ref[...] = jnp.broadcast_to(inv[:, None], (_BN, 8))
        h1 = ((p_ref[0] + p_ref[1]) * inv[:, None]
              + jnp.dot(h_ref[...], w_ref[...],
                        preferred_element_type=jnp.float32) + b_ref[...])
        h1_ref[...] = h1
        for r in range(NUM_RELS):
            o_ref[r] = jnp.dot(h1, w2_ref[r], preferred_element_type=jnp.float32)

    return pl.pallas_call(
        body,
        grid=(NROW // _BN,),
        in_specs=[
            pl.BlockSpec((NC, _BN, D), lambda n: (0, n, 0)),
            pl.BlockSpec((NC, _BN, D), lambda n: (0, n, 0)),
            pl.BlockSpec((_BN, D), lambda n: (n, 0)),
            pl.BlockSpec((D, D), lambda n: (0, 0)),
            pl.BlockSpec((1, D), lambda n: (0, 0)),
            pl.BlockSpec((NUM_RELS, D, D), lambda n: (0, 0, 0)),
        ],
        out_specs=[
            pl.BlockSpec((_BN, D), lambda n: (n, 0)),
            pl.BlockSpec((NUM_RELS, _BN, D), lambda n: (0, n, 0)),
            pl.BlockSpec((_BN, 8), lambda n: (n, 0)),
        ],
        out_shape=[
            jax.ShapeDtypeStruct((NROW, D), jnp.float32),
            jax.ShapeDtypeStruct((NUM_RELS, NROW, D), jnp.float32),
            jax.ShapeDtypeStruct((NROW, 8), jnp.float32),
        ],
    )(p, deg, hprev, loopw, bias.reshape(1, D), w2)


def _tc_gru(q, inv8, h1, loopw, bias, h0, w_iht, w_hht, b_ih, b_hh):
    def body(q_ref, i_ref, h1_ref, w_ref, b_ref, h0_ref, wi_ref, wh_ref,
             bi_ref, bh_ref, o_ref):
        inv = i_ref[...][:, :1]
        x = ((q_ref[0] + q_ref[1]) * inv
             + jnp.dot(h1_ref[...], w_ref[...],
                       preferred_element_type=jnp.float32) + b_ref[...])
        h0 = h0_ref[...]
        gi = jnp.dot(x, wi_ref[...], preferred_element_type=jnp.float32) + bi_ref[...]
        gh = jnp.dot(h0, wh_ref[...], preferred_element_type=jnp.float32) + bh_ref[...]
        r = jax.nn.sigmoid(gi[:, :D] + gh[:, :D])
        z = jax.nn.sigmoid(gi[:, D:2 * D] + gh[:, D:2 * D])
        n = jnp.tanh(gi[:, 2 * D:] + r * gh[:, 2 * D:])
        o_ref[...] = ((1.0 - z) * n + z * h0)[:, None, :]

    return pl.pallas_call(
        body,
        grid=(NROW // _BN,),
        in_specs=[
            pl.BlockSpec((NC, _BN, D), lambda n: (0, n, 0)),
            pl.BlockSpec((_BN, 8), lambda n: (n, 0)),
            pl.BlockSpec((_BN, D), lambda n: (n, 0)),
            pl.BlockSpec((D, D), lambda n: (0, 0)),
            pl.BlockSpec((1, D), lambda n: (0, 0)),
            pl.BlockSpec((_BN, D), lambda n: (n, 0)),
            pl.BlockSpec((D, 3 * D), lambda n: (0, 0)),
            pl.BlockSpec((D, 3 * D), lambda n: (0, 0)),
            pl.BlockSpec((1, 3 * D), lambda n: (0, 0)),
            pl.BlockSpec((1, 3 * D), lambda n: (0, 0)),
        ],
        out_specs=pl.BlockSpec((_BN, 1, D), lambda n: (n, 0, 0)),
        out_shape=jax.ShapeDtypeStruct((N_NODES, 1, D), jnp.float32),
    )(q, inv8, h1, loopw, bias.reshape(1, D), h0, w_iht, w_hht,
      b_ih.reshape(1, 3 * D), b_hh.reshape(1, 3 * D))


@jax.jit
def kernel(edge_index, rel_type, nid, static_structural, dynamic_structural,
           coeff1, bases1, loop1, bias1, coeff2, bases2, loop2, bias2,
           gru_w_ih, gru_w_hh, gru_b_ih, gru_b_hh):
    gidx2 = _tc_gather_idx(edge_index[0], rel_type).reshape(NW, EPT)
    dst3 = edge_index[1].reshape(NW, NCH, K)
    nid2 = jnp.concatenate(
        [nid, jnp.zeros((NROW - N_NODES,), jnp.int32)]).reshape(NW, GPT)
    dyn_flat = dynamic_structural.reshape(-1, D)

    h, h0 = _sc_entity_gather(nid2, static_structural, dyn_flat)

    w1 = _tc_combine(coeff1, bases1)
    ht1 = _tc_relmm(h, w1).reshape(NUM_RELS * NROW, D)
    deg16 = _sc_degree(dst3)
    p1 = _sc_edge_agg(ht1, gidx2, dst3)

    w2 = _tc_combine(coeff2, bases2)
    h1, ht2, inv8 = _tc_mix_relmm(p1, deg16, h, loop1, bias1, w2)
    ht2 = ht2.reshape(NUM_RELS * NROW, D)
    p2 = _sc_edge_agg(ht2, gidx2, dst3)

    return _tc_gru(p2, inv8, h1, loop2, bias2, h0,
                   gru_w_ih.T, gru_w_hh.T, gru_b_ih, gru_b_hh)


# speedup vs baseline: 9.8215x; 1.0292x over previous
import functools

import jax
import jax.numpy as jnp
from jax import lax
from jax.experimental import pallas as pl
from jax.experimental.pallas import tpu as pltpu
from jax.experimental.pallas import tpu_sc as plsc

N_NODES = 10000
NUM_RELS = 16
N_EDGES = 320000
D = 128

NC = 2
NS = 16
NW = NC * NS

NROW = 10240
SLAB = NROW // NS
GPT = NROW // NW
EPT = N_EDGES // NW
K = 80
NCH = EPT // K

_mesh = lambda: plsc.VectorSubcoreMesh(
    core_axis_name="c", subcore_axis_name="s", num_cores=NC, num_subcores=NS)


def _sc_entity_gather(nid2, stat, dyn):
    @functools.partial(
        pl.kernel,
        out_type=[jax.ShapeDtypeStruct((NROW, D), jnp.float32),
                  jax.ShapeDtypeStruct((NROW, D), jnp.float32)],
        mesh=_mesh(),
        scratch_types=[
            pltpu.VMEM((GPT,), jnp.int32),
            pltpu.VMEM((K, D), jnp.float32),
            pltpu.VMEM((K, D), jnp.float32),
            pltpu.SemaphoreType.DMA,
            pltpu.SemaphoreType.DMA,
        ],
    )
    def k(nid_h, stat_h, dyn_h, h_o, h0_o, idx_v, rows_v, rows2_v, sem, sem2):
        wid = lax.axis_index("c") * NS + lax.axis_index("s")
        base = wid * GPT
        pltpu.sync_copy(nid_h.at[wid], idx_v)

        def body(i, carry):
            off = base + i * K
            cp1 = pltpu.async_copy(stat_h.at[idx_v.at[pl.ds(i * K, K)]], rows_v, sem)
            cp2 = pltpu.async_copy(dyn_h.at[idx_v.at[pl.ds(i * K, K)]], rows2_v, sem2)
            cp1.wait()
            pltpu.sync_copy(rows_v, h_o.at[pl.ds(off, K)])
            cp2.wait()
            pltpu.sync_copy(rows2_v, h0_o.at[pl.ds(off, K)])
            return carry

        lax.fori_loop(0, GPT // K, body, 0)

    return k(nid2, stat, dyn)


def _sc_edge_agg(ht_flat, gidx2, dst3):
    @functools.partial(
        pl.kernel,
        out_type=jax.ShapeDtypeStruct((NC, NROW, D), jnp.float32),
        mesh=_mesh(),
        scratch_types=[
            pltpu.VMEM((EPT,), jnp.int32),
            pltpu.VMEM((NCH, K), jnp.int32),
            pltpu.VMEM((K, D), jnp.float32),
            pltpu.VMEM((K, D), jnp.float32),
            pltpu.VMEM_SHARED((NROW, D), jnp.float32),
            pltpu.SemaphoreType.DMA,
            pltpu.SemaphoreType.DMA,
        ],
    )
    def k(ht, gidx_h, dst_h, out, idx2, edst2, rows0, rows1, acc, sem0, sem1):
        c = lax.axis_index("c")
        s = lax.axis_index("s")
        wid = c * NS + s
        zero16 = jnp.zeros((16,), jnp.float32)

        def zrow(i, carry):
            for j in range(D // 16):
                rows0[i, pl.ds(j * 16, 16)] = zero16
            return carry
        lax.fori_loop(0, K, zrow, 0)

        def zcp(i, carry):
            pltpu.sync_copy(rows0, acc.at[pl.ds(s * SLAB + i * K, K)])
            return carry
        lax.fori_loop(0, SLAB // K, zcp, 0)

        pltpu.sync_copy(gidx_h.at[wid], idx2)
        pltpu.sync_copy(dst_h.at[wid], edst2)

        plsc.subcore_barrier()

        pltpu.async_copy(ht.at[idx2.at[pl.ds(0, K)]], rows0, sem0)

        def body(t, carry):
            a = 2 * t
            pltpu.async_copy(ht.at[idx2.at[pl.ds((a + 1) * K, K)]], rows1, sem1)
            pltpu.make_async_copy(ht.at[idx2.at[pl.ds(a * K, K)]], rows0, sem0).wait()
            pltpu.sync_copy(rows0, acc.at[edst2.at[a]], add=True)
            pltpu.async_copy(ht.at[idx2.at[pl.ds((a + 2) * K, K)]], rows0, sem0)
            pltpu.make_async_copy(ht.at[idx2.at[pl.ds((a + 1) * K, K)]], rows1, sem1).wait()
            pltpu.sync_copy(rows1, acc.at[edst2.at[a + 1]], add=True)
            return carry

        lax.fori_loop(0, (NCH - 1) // 2, body, 0)
        pltpu.make_async_copy(ht.at[idx2.at[pl.ds((NCH - 1) * K, K)]], rows0, sem0).wait()
        pltpu.sync_copy(rows0, acc.at[edst2.at[NCH - 1]], add=True)

        plsc.subcore_barrier()

        pltpu.sync_copy(acc.at[pl.ds(s * SLAB, SLAB)],
                        out.at[c, pl.ds(s * SLAB, SLAB)])

    return k(ht_flat, gidx2, dst3)


def _sc_degree(dst3):
    @functools.partial(
        pl.kernel,
        out_type=jax.ShapeDtypeStruct((NC, NROW, D), jnp.float32),
        mesh=_mesh(),
        scratch_types=[
            pltpu.VMEM((NCH, K), jnp.int32),
            pltpu.VMEM((K, D), jnp.float32),
            pltpu.VMEM_SHARED((NROW, D), jnp.float32),
        ],
    )
    def k(dst_h, degout, edst2, ones_v, deg_sh):
        c = lax.axis_index("c")
        s = lax.axis_index("s")
        wid = c * NS + s
        zero16 = jnp.zeros((16,), jnp.float32)
        one16 = jnp.ones((16,), jnp.float32)

        def zdrow(i, carry):
            for j in range(D // 16):
                ones_v[i, pl.ds(j * 16, 16)] = zero16
            return carry
        lax.fori_loop(0, K, zdrow, 0)

        def zdcp(i, carry):
            pltpu.sync_copy(ones_v, deg_sh.at[pl.ds(s * SLAB + i * K, K)])
            return carry
        lax.fori_loop(0, SLAB // K, zdcp, 0)

        def onesrow(i, carry):
            ones_v[i, pl.ds(0, 16)] = one16
            return carry
        lax.fori_loop(0, K, onesrow, 0)

        pltpu.sync_copy(dst_h.at[wid], edst2)

        plsc.subcore_barrier()

        def body(i, carry):
            pltpu.sync_copy(ones_v, deg_sh.at[edst2.at[i]], add=True)
            return carry

        lax.fori_loop(0, NCH, body, 0)

        plsc.subcore_barrier()

        pltpu.sync_copy(deg_sh.at[pl.ds(s * SLAB, SLAB)],
                        degout.at[c, pl.ds(s * SLAB, SLAB)])

    return k(dst3)


def _tc_gather_idx(src, rel):
    def body(s_ref, r_ref, o_ref):
        o_ref[...] = r_ref[...] * NROW + s_ref[...]

    return pl.pallas_call(
        body,
        out_shape=jax.ShapeDtypeStruct((N_EDGES // 128, 128), jnp.int32),
    )(src.reshape(N_EDGES // 128, 128),
      rel.reshape(N_EDGES // 128, 128)).reshape(N_EDGES)

def _tc_combine(coeff, bases):
    nb = bases.shape[0]
    bases_f = bases.reshape(nb, D * D)

    def body(c_ref, b_ref, o_ref):
        o_ref[...] = jnp.dot(c_ref[...], b_ref[...],
                             preferred_element_type=jnp.float32)

    w = pl.pallas_call(
        body,
        out_shape=jax.ShapeDtypeStruct((NUM_RELS, D * D), jnp.float32),
    )(coeff, bases_f)
    return w.reshape(NUM_RELS, D, D)


_BN = 2048


def _tc_relmm(h, w):
    def body(h_ref, w_ref, o_ref):
        hb = h_ref[...]
        for r in range(NUM_RELS):
            o_ref[r] = jnp.dot(hb, w_ref[r], preferred_element_type=jnp.float32)

    return pl.pallas_call(
        body,
        grid=(NROW // _BN,),
        in_specs=[
            pl.BlockSpec((_BN, D), lambda n: (n, 0)),
            pl.BlockSpec((NUM_RELS, D, D), lambda n: (0, 0, 0)),
        ],
        out_specs=pl.BlockSpec((NUM_RELS, _BN, D), lambda n: (0, n, 0)),
        out_shape=jax.ShapeDtypeStruct((NUM_RELS, NROW, D), jnp.float32),
    )(h, w)


def _tc_mix_relmm(p, deg, hprev, loopw, bias, w2):
    def body(p_ref, d_ref, h_ref, w_ref, b_ref, w2_ref, h1_ref, o_ref,
             inv_ref):
        dg = d_ref[0, :, 0] + d_ref[1, :, 0]
        inv = 1.0 / jnp.maximum(dg, 1.0)
        inv_ref[...] = jnp.broadcast_to(inv[:, None], (_BN, 8))
        h1 = ((p_ref[0] + p_ref[1]) * inv[:, None]
              + jnp.dot(h_ref[...], w_ref[...],
                        preferred_element_type=jnp.float32) + b_ref[...])
        h1_ref[...] = h1
        for r in range(NUM_RELS):
            o_ref[r] = jnp.dot(h1, w2_ref[r], preferred_element_type=jnp.float32)

    return pl.pallas_call(
        body,
        grid=(NROW // _BN,),
        in_specs=[
            pl.BlockSpec((NC, _BN, D), lambda n: (0, n, 0)),
            pl.BlockSpec((NC, _BN, D), lambda n: (0, n, 0)),
            pl.BlockSpec((_BN, D), lambda n: (n, 0)),
            pl.BlockSpec((D, D), lambda n: (0, 0)),
            pl.BlockSpec((1, D), lambda n: (0, 0)),
            pl.BlockSpec((NUM_RELS, D, D), lambda n: (0, 0, 0)),
        ],
        out_specs=[
            pl.BlockSpec((_BN, D), lambda n: (n, 0)),
            pl.BlockSpec((NUM_RELS, _BN, D), lambda n: (0, n, 0)),
            pl.BlockSpec((_BN, 8), lambda n: (n, 0)),
        ],
        out_shape=[
            jax.ShapeDtypeStruct((NROW, D), jnp.float32),
            jax.ShapeDtypeStruct((NUM_RELS, NROW, D), jnp.float32),
            jax.ShapeDtypeStruct((NROW, 8), jnp.float32),
        ],
    )(p, deg, hprev, loopw, bias.reshape(1, D), w2)


def _tc_gru(q, inv8, h1, loopw, bias, h0, w_iht, w_hht, b_ih, b_hh):
    def body(q_ref, i_ref, h1_ref, w_ref, b_ref, h0_ref, wi_ref, wh_ref,
             bi_ref, bh_ref, o_ref):
        inv = i_ref[...][:, :1]
        x = ((q_ref[0] + q_ref[1]) * inv
             + jnp.dot(h1_ref[...], w_ref[...],
                       preferred_element_type=jnp.float32) + b_ref[...])
        h0 = h0_ref[...]
        gi = jnp.dot(x, wi_ref[...], preferred_element_type=jnp.float32) + bi_ref[...]
        gh = jnp.dot(h0, wh_ref[...], preferred_element_type=jnp.float32) + bh_ref[...]
        r = jax.nn.sigmoid(gi[:, :D] + gh[:, :D])
        z = jax.nn.sigmoid(gi[:, D:2 * D] + gh[:, D:2 * D])
        n = jnp.tanh(gi[:, 2 * D:] + r * gh[:, 2 * D:])
        o_ref[...] = ((1.0 - z) * n + z * h0)[:, None, :]

    return pl.pallas_call(
        body,
        grid=(NROW // _BN,),
        in_specs=[
            pl.BlockSpec((NC, _BN, D), lambda n: (0, n, 0)),
            pl.BlockSpec((_BN, 8), lambda n: (n, 0)),
            pl.BlockSpec((_BN, D), lambda n: (n, 0)),
            pl.BlockSpec((D, D), lambda n: (0, 0)),
            pl.BlockSpec((1, D), lambda n: (0, 0)),
            pl.BlockSpec((_BN, D), lambda n: (n, 0)),
            pl.BlockSpec((D, 3 * D), lambda n: (0, 0)),
            pl.BlockSpec((D, 3 * D), lambda n: (0, 0)),
            pl.BlockSpec((1, 3 * D), lambda n: (0, 0)),
            pl.BlockSpec((1, 3 * D), lambda n: (0, 0)),
        ],
        out_specs=pl.BlockSpec((_BN, 1, D), lambda n: (n, 0, 0)),
        out_shape=jax.ShapeDtypeStruct((N_NODES, 1, D), jnp.float32),
    )(q, inv8, h1, loopw, bias.reshape(1, D), h0, w_iht, w_hht,
      b_ih.reshape(1, 3 * D), b_hh.reshape(1, 3 * D))


@jax.jit
def kernel(edge_index, rel_type, nid, static_structural, dynamic_structural,
           coeff1, bases1, loop1, bias1, coeff2, bases2, loop2, bias2,
           gru_w_ih, gru_w_hh, gru_b_ih, gru_b_hh):
    gidx2 = _tc_gather_idx(edge_index[0], rel_type).reshape(NW, EPT)
    dst3 = edge_index[1].reshape(NW, NCH, K)
    nid2 = jnp.concatenate(
        [nid, jnp.zeros((NROW - N_NODES,), jnp.int32)]).reshape(NW, GPT)
    dyn_flat = dynamic_structural.reshape(-1, D)

    h, h0 = _sc_entity_gather(nid2, static_structural, dyn_flat)

    w1 = _tc_combine(coeff1, bases1)
    ht1 = _tc_relmm(h, w1).reshape(NUM_RELS * NROW, D)
    deg16 = _sc_degree(dst3)
    p1 = _sc_edge_agg(ht1, gidx2, dst3)

    w2 = _tc_combine(coeff2, bases2)
    h1, ht2, inv8 = _tc_mix_relmm(p1, deg16, h, loop1, bias1, w2)
    ht2 = ht2.reshape(NUM_RELS * NROW, D)
    p2 = _sc_edge_agg(ht2, gidx2, dst3)

    return _tc_gru(p2, inv8, h1, loop2, bias2, h0,
                   gru_w_ih.T, gru_w_hh.T, gru_b_ih, gru_b_hh)
